# Initial kernel scaffold; baseline (speedup 1.0000x reference)
import jax
import jax.numpy as jnp
from jax.experimental import pallas as pl


def kernel(edge_index, h, W_att, W_phi, b_phi, W_p, W_pp, W_f, b_f, W_self, b_self, W_A, b_A, W_str, b_str, ln_w, ln_b):
    raise NotImplementedError("write your pallas kernel here")


# speedup vs baseline: 3.5295x; 3.5295x over previous
import jax
import jax.numpy as jnp
from jax import lax
from jax.experimental import pallas as pl
from jax.experimental.pallas import tpu as pltpu
from jax.experimental.pallas import tpu_sc as plsc

N = 10000
E = 320000
D = 128
S = 7

NC = 2
NS = 16
NW = NC * NS
EPW = E // NW
NPAD = 10240
RPT = NPAD // NS

CH = 80
NCH = EPW // CH
NG = CH // 16

BN = 1000
GRID = N // BN

F32 = jnp.float32
I32 = jnp.int32

_SC_PARAMS = pltpu.CompilerParams(use_tc_tiling_on_sc=False,
                                  needs_layout_passes=False)


def _tc1_body(x_ref, wbig_ref, bphi_ref, bf_ref, src_ref, tgt_ref, phi_ref):
    x = x_ref[...]
    y = jnp.dot(x, wbig_ref[...], preferred_element_type=F32)
    src_ref[...] = y[:, 0:256]
    tgt_ref[:, 0:128] = x
    tgt_ref[:, 128:256] = y[:, 256:384]
    tgt_ref[:, 256:272] = jnp.exp(-(y[:, 512:528] + bf_ref[...]))
    phi_ref[...] = y[:, 384:512] + bphi_ref[...]


def _tc1(h, wbig, bphi, bf16c):
    return pl.pallas_call(
        _tc1_body,
        grid=(GRID,),
        in_specs=[
            pl.BlockSpec((BN, D), lambda i: (i, 0)),
            pl.BlockSpec((D, 528), lambda i: (0, 0)),
            pl.BlockSpec((1, D), lambda i: (0, 0)),
            pl.BlockSpec((1, 16), lambda i: (0, 0)),
        ],
        out_specs=[
            pl.BlockSpec((BN, 256), lambda i: (i, 0)),
            pl.BlockSpec((BN, 272), lambda i: (i, 0)),
            pl.BlockSpec((BN, D), lambda i: (i, 0)),
        ],
        out_shape=[
            jax.ShapeDtypeStruct((N, 256), F32),
            jax.ShapeDtypeStruct((N, 272), F32),
            jax.ShapeDtypeStruct((N, D), F32),
        ],
    )(h, wbig, bphi, bf16c)


def _sc1_body(src_tab, tgt_tab, idx_hbm,
              es_hbm, accA_out, accB_out, accE_out,
              idxp, srows, trows, es_all, accA, accB, accE,
              semI, semR):
    cid = lax.axis_index("c")
    sid = lax.axis_index("s")
    wid = sid * NC + cid

    z16 = jnp.zeros((16,), F32)
    lanes = lax.iota(I32, 16)

    def _zacc(i, _):
        sl = pl.ds(i * 16, 16)
        accA[sl] = z16
        accB[sl] = z16
        accE[sl] = z16
        return 0

    lax.fori_loop(0, NPAD // 16, _zacc, 0)

    def _idx_issue(k, r):
        return pltpu.async_copy(idx_hbm.at[wid, k], idxp[r], semI[r])

    def _rows_issue(k, b, r):
        a = pltpu.async_copy(src_tab.at[idxp[r].at[0]], srows[b], semR[b])
        c = pltpu.async_copy(tgt_tab.at[idxp[r].at[1]], trows[b], semR[b])
        return a, c

    def _rows_wait(k, b, r):
        pltpu.make_async_copy(src_tab.at[idxp[r].at[0]], srows[b],
                              semR[b]).wait()
        pltpu.make_async_copy(tgt_tab.at[idxp[r].at[1]], trows[b],
                              semR[b]).wait()

    def _idx_wait(k, r):
        pltpu.make_async_copy(idx_hbm.at[wid, k], idxp[r], semI[r]).wait()

    def _compute(k, b, r):
        for g in range(NG):
            eids = lanes + (g * 16)

            def _dot(do, carry):
                sa, pa = carry
                d0 = do * 8
                for j in range(8):
                    cd = jnp.full((16,), d0 + j, I32)
                    cd2 = cd + 128
                    a1 = plsc.load_gather(srows[b], [eids, cd])
                    b1 = plsc.load_gather(trows[b], [eids, cd])
                    a2 = plsc.load_gather(srows[b], [eids, cd2])
                    b2 = plsc.load_gather(trows[b], [eids, cd2])
                    sa = sa + a1 * b1
                    pa = pa + a2 * b2
                return (sa, pa)

            sacc, pacc = lax.fori_loop(0, 16, _dot, (z16, z16))
            en = plsc.load_gather(trows[b], [eids, jnp.full((16,), 256, I32)])
            es = jnp.exp(sacc)
            ep = jnp.exp(pacc)
            epn = ep * en
            es_all[k, pl.ds(g * 16, 16)] = es
            gsl = pl.ds(g * 16, 16)
            srcv = idxp[r][0, gsl]
            tgtv = idxp[r][1, gsl]
            plsc.addupdate_scatter(accA, [tgtv], es)
            plsc.addupdate_scatter(accB, [srcv], ep)
            plsc.addupdate_scatter(accE, [srcv], epn)

    def _body(k, b, r):
        @pl.when(k + 1 < NCH)
        def _():
            _idx_wait(k + 1, (r + 1) % 3)
            _rows_issue(k + 1, (b + 1) % 2, (r + 1) % 3)

        @pl.when(k + 2 < NCH)
        def _():
            _idx_issue(k + 2, (r + 2) % 3)

        @pl.when(k < NCH)
        def _():
            _rows_wait(k, b, r)
            _compute(k, b, r)

    _idx_issue(0, 0)
    _idx_issue(1, 1)
    _idx_wait(0, 0)
    _rows_issue(0, 0, 0)

    def _super(gi, _):
        k0 = gi * 6
        for j in range(6):
            _body(k0 + j, j % 2, j % 3)
        return 0

    lax.fori_loop(0, (NCH + 6) // 6, _super, 0)

    pltpu.sync_copy(es_all, es_hbm.at[wid])
    pltpu.sync_copy(accA, accA_out.at[cid, sid])
    pltpu.sync_copy(accB, accB_out.at[cid, sid])
    pltpu.sync_copy(accE, accE_out.at[cid, sid])


def _sc1(src_tab, tgt_tab, idx_packed):
    mesh = plsc.VectorSubcoreMesh(core_axis_name="c", subcore_axis_name="s")
    return pl.kernel(
        _sc1_body,
        out_type=[
            jax.ShapeDtypeStruct((NW, NCH, CH), F32),
            jax.ShapeDtypeStruct((NC, NS, NPAD), F32),
            jax.ShapeDtypeStruct((NC, NS, NPAD), F32),
            jax.ShapeDtypeStruct((NC, NS, NPAD), F32),
        ],
        mesh=mesh,
        compiler_params=_SC_PARAMS,
        scratch_types=[
            [pltpu.VMEM((2, CH), I32) for _ in range(3)],
            [pltpu.VMEM((CH, 256), F32) for _ in range(2)],
            [pltpu.VMEM((CH, 272), F32) for _ in range(2)],
            pltpu.VMEM((NCH, CH), F32),
            pltpu.VMEM((NPAD,), F32),
            pltpu.VMEM((NPAD,), F32),
            pltpu.VMEM((NPAD,), F32),
            [pltpu.SemaphoreType.DMA for _ in range(3)],
            [pltpu.SemaphoreType.DMA for _ in range(2)],
        ],
    )(src_tab, tgt_tab, idx_packed)


def _tc2_body(accb_ref, acce_ref, phi_ref, phic_ref):
    bden = jnp.sum(accb_ref[...], axis=1, keepdims=True)
    sedge = jnp.sum(acce_ref[...], axis=1, keepdims=True)
    t = sedge / (bden + 1e-9) + 1e-8
    coef = t / (t + 0.6065306597126334)
    phic_ref[...] = phi_ref[...] * coef


def _tc2(acc_b, acc_e, phi):
    return pl.pallas_call(
        _tc2_body,
        grid=(GRID,),
        in_specs=[
            pl.BlockSpec((BN, NW), lambda i: (i, 0)),
            pl.BlockSpec((BN, NW), lambda i: (i, 0)),
            pl.BlockSpec((BN, D), lambda i: (i, 0)),
        ],
        out_specs=pl.BlockSpec((BN, D), lambda i: (i, 0)),
        out_shape=jax.ShapeDtypeStruct((N, D), F32),
    )(acc_b, acc_e, phi)


def _sc2_body(phic_hbm, idx_hbm, es_hbm,
              matt_out,
              matt_sh, idxp, es_v, prows, zb2,
              semI, semE, semR, semO):
    cid = lax.axis_index("c")
    sid = lax.axis_index("s")
    wid = sid * NC + cid

    z16 = jnp.zeros((16,), F32)
    lanes = lax.iota(I32, 16)

    def _zrow(i, _):
        for j in range(8):
            zb2[i, pl.ds(j * 16, 16)] = z16
        return 0

    lax.fori_loop(0, 64, _zrow, 0)
    for k in range(10):
        pltpu.sync_copy(zb2, matt_sh.at[pl.ds(sid * RPT + k * 64, 64)])
    plsc.subcore_barrier()

    def _idx_issue(k, r):
        pltpu.async_copy(idx_hbm.at[wid, k], idxp[r], semI[r])
        pltpu.async_copy(es_hbm.at[wid, k], es_v[r], semE[r])

    def _idx_wait(k, r):
        pltpu.make_async_copy(idx_hbm.at[wid, k], idxp[r], semI[r]).wait()
        pltpu.make_async_copy(es_hbm.at[wid, k], es_v[r], semE[r]).wait()

    def _rows_issue(k, b, r):
        pltpu.async_copy(phic_hbm.at[idxp[r].at[0]], prows[b], semR[b])

    def _rows_wait(k, b, r):
        pltpu.make_async_copy(phic_hbm.at[idxp[r].at[0]], prows[b],
                              semR[b]).wait()

    def _scat_issue(k, b, r):
        pltpu.async_copy(prows[b], matt_sh.at[idxp[r].at[1]], semO[b],
                         add=True)

    def _scat_wait(k, b, r):
        pltpu.make_async_copy(prows[b], matt_sh.at[idxp[r].at[1]],
                              semO[b]).wait()

    def _compute(k, b, r):
        for g in range(NG):
            eids = lanes + (g * 16)
            w = es_v[r][pl.ds(g * 16, 16)]

            def _scale(do, _c):
                d0 = do * 8
                for j in range(8):
                    cd = jnp.full((16,), d0 + j, I32)
                    col = plsc.load_gather(prows[b], [eids, cd]) * w
                    plsc.store_scatter(prows[b], [eids, cd], col)
                return 0

            lax.fori_loop(0, 16, _scale, 0)

    def _body(k, b, r):
        @pl.when(k + 1 < NCH)
        def _():
            @pl.when(k >= 1)
            def _():
                _scat_wait(k - 1, (b + 1) % 2, (r + 2) % 3)

            _idx_wait(k + 1, (r + 1) % 3)
            _rows_issue(k + 1, (b + 1) % 2, (r + 1) % 3)

            @pl.when(k + 2 < NCH)
            def _():
                _idx_issue(k + 2, (r + 2) % 3)

        @pl.when(k < NCH)
        def _():
            _rows_wait(k, b, r)
            _compute(k, b, r)
            _scat_issue(k, b, r)

    _idx_issue(0, 0)
    _idx_issue(1, 1)
    _idx_wait(0, 0)
    _rows_issue(0, 0, 0)

    def _super(gi, _):
        k0 = gi * 6
        for j in range(6):
            _body(k0 + j, j % 2, j % 3)
        return 0

    lax.fori_loop(0, (NCH + 6) // 6, _super, 0)

    _scat_wait(NCH - 2, (NCH - 2) % 2, (NCH - 2) % 3)
    _scat_wait(NCH - 1, (NCH - 1) % 2, (NCH - 1) % 3)
    plsc.subcore_barrier()

    rs = pl.ds(sid * RPT, RPT)
    pltpu.sync_copy(matt_sh.at[rs], matt_out.at[cid, rs])


def _sc2(phic, idx_packed, es):
    mesh = plsc.VectorSubcoreMesh(core_axis_name="c", subcore_axis_name="s")
    return pl.kernel(
        _sc2_body,
        out_type=jax.ShapeDtypeStruct((NC, NPAD, D), F32),
        mesh=mesh,
        compiler_params=_SC_PARAMS,
        scratch_types=[
            pltpu.VMEM_SHARED((NPAD, D), F32),
            [pltpu.VMEM((2, CH), I32) for _ in range(3)],
            [pltpu.VMEM((CH,), F32) for _ in range(3)],
            [pltpu.VMEM((CH, D), F32) for _ in range(2)],
            pltpu.VMEM((64, D), F32),
            [pltpu.SemaphoreType.DMA for _ in range(3)],
            [pltpu.SemaphoreType.DMA for _ in range(3)],
            [pltpu.SemaphoreType.DMA for _ in range(2)],
            [pltpu.SemaphoreType.DMA for _ in range(2)],
        ],
    )(phic, idx_packed, es)


def _tc3_body(x_ref, matt_ref, acca_ref, wc_ref, wa_ref, bsum_ref,
              lnw_ref, lnb_ref, out_ref):
    x = x_ref[...]
    at = jnp.sum(acca_ref[...], axis=1, keepdims=True)
    ia = 1.0 / (at + 1e-9)
    m = (matt_ref[0] + matt_ref[1]) * ia
    o = (jnp.dot(x, wc_ref[...], preferred_element_type=F32)
         + jnp.dot(m, wa_ref[...], preferred_element_type=F32)
         + bsum_ref[...])
    o = jnp.maximum(o, 0.0) + x
    mu = jnp.mean(o, axis=-1, keepdims=True)
    var = jnp.mean((o - mu) * (o - mu), axis=-1, keepdims=True)
    out_ref[...] = (o - mu) * lax.rsqrt(var + 1e-5) * lnw_ref[...] + lnb_ref[...]


def _tc3(h, matt, acc_a, wc, wa, bsum, lnw, lnb):
    return pl.pallas_call(
        _tc3_body,
        grid=(GRID,),
        in_specs=[
            pl.BlockSpec((BN, D), lambda i: (i, 0)),
            pl.BlockSpec((NC, BN, D), lambda i: (0, i, 0)),
            pl.BlockSpec((BN, NW), lambda i: (i, 0)),
            pl.BlockSpec((D, D), lambda i: (0, 0)),
            pl.BlockSpec((D, D), lambda i: (0, 0)),
            pl.BlockSpec((1, D), lambda i: (0, 0)),
            pl.BlockSpec((1, D), lambda i: (0, 0)),
            pl.BlockSpec((1, D), lambda i: (0, 0)),
        ],
        out_specs=pl.BlockSpec((BN, D), lambda i: (i, 0)),
        out_shape=jax.ShapeDtypeStruct((N, D), F32),
    )(h, matt, acc_a, wc, wa, bsum, lnw, lnb)


def kernel(edge_index, h, W_att, W_phi, b_phi, W_p, W_pp, W_f, b_f,
           W_self, b_self, W_A, b_A, W_str, b_str, ln_w, ln_b):
    src = edge_index[0].astype(I32)
    tgt = edge_index[1].astype(I32)
    idx_packed = jnp.stack(
        [src.reshape(NW, NCH, CH), tgt.reshape(NW, NCH, CH)], axis=2)

    k6 = jnp.arange(S - 1)
    watt2 = W_att.at[k6, k6].add(1.0)
    wf_pad = jnp.zeros((D, 16), F32).at[:, 0].set(W_f[:, 0])
    wbig = jnp.concatenate([watt2, W_p, W_pp, W_phi, wf_pad], axis=1)
    bf16c = jnp.broadcast_to(b_f.reshape(1, 1), (1, 16))

    src_tab, tgt_tab, phi = _tc1(h, wbig, b_phi.reshape(1, D), bf16c)

    es, acc_a, acc_b, acc_e = _sc1(src_tab, tgt_tab, idx_packed)
    acc_a = acc_a.reshape(NW, NPAD).T[:N]
    acc_b = acc_b.reshape(NW, NPAD).T[:N]
    acc_e = acc_e.reshape(NW, NPAD).T[:N]

    phic = _tc2(acc_b, acc_e, phi)

    matt = _sc2(phic, idx_packed, es)[:, :N]

    wstr_pad = jnp.zeros((D, D), F32).at[:S - 1, :].set(W_str)
    wc = W_self + wstr_pad
    bsum = (b_self + b_A + b_str).reshape(1, D)

    return _tc3(h, matt, acc_a, wc, W_A, bsum,
                ln_w.reshape(1, D), ln_b.reshape(1, D))
